# Initial kernel scaffold; baseline (speedup 1.0000x reference)
import jax
import jax.numpy as jnp
from jax.experimental import pallas as pl


def kernel(features, current, neigh_idx, W, b, gamma, beta):
    raise NotImplementedError("write your pallas kernel here")


# speedup vs baseline: 3.3891x; 3.3891x over previous
import functools

import jax
import jax.numpy as jnp
from jax import lax
from jax.experimental import pallas as pl
from jax.experimental.pallas import tpu as pltpu
from jax.experimental.pallas import tpu_sc as plsc

N_TOTAL = 100000
N_OUT = 25000
D = 128
S = 25
R = S + 1
NC, NS = 2, 16
NW = NC * NS
BPW = 784
N_PAD = NW * BPW
C = 16
NCHUNK = BPW // C


def _sc_gather(features, idx_flat):
    mesh = plsc.VectorSubcoreMesh(
        core_axis_name="c", subcore_axis_name="s",
        num_cores=NC, num_subcores=NS)

    @functools.partial(
        pl.kernel,
        out_type=(jax.ShapeDtypeStruct((N_PAD * D,), jnp.float32),
                  jax.ShapeDtypeStruct((N_PAD * D,), jnp.float32)),
        mesh=mesh,
        scratch_types=[
            pltpu.VMEM((R * C,), jnp.int32),
            pltpu.VMEM((R * C, D), jnp.float32),
            pltpu.VMEM((C * D,), jnp.float32),
            pltpu.VMEM((C * D,), jnp.float32),
            pltpu.SemaphoreType.DMA,
        ],
    )
    def k(feat_hbm, idx_hbm, self_hbm, sum_hbm, idx_v, gbuf, sbuf, mbuf, sem):
        wid = lax.axis_index("s") * NC + lax.axis_index("c")
        base = wid * BPW

        def chunk(kc, carry):
            c0 = base + kc * C
            pltpu.sync_copy(idx_hbm.at[pl.ds(c0 * R, R * C)], idx_v)
            pltpu.async_copy(feat_hbm.at[idx_v], gbuf, sem).wait()

            def node(c, carry2):
                r0 = c * R
                for u in range(8):
                    sbuf[pl.ds(c * D + 16 * u, 16)] = gbuf[r0, pl.ds(16 * u, 16)]
                for u in range(8):
                    acc = gbuf[r0 + 1, pl.ds(16 * u, 16)]
                    for j in range(2, R):
                        acc = acc + gbuf[r0 + j, pl.ds(16 * u, 16)]
                    mbuf[pl.ds(c * D + 16 * u, 16)] = acc
                return carry2

            lax.fori_loop(0, C, node, 0)
            pltpu.sync_copy(sbuf, self_hbm.at[pl.ds(c0 * D, C * D)])
            pltpu.sync_copy(mbuf, sum_hbm.at[pl.ds(c0 * D, C * D)])
            return carry

        lax.fori_loop(0, NCHUNK, chunk, 0)

    return k(features, idx_flat)


BR = 1000
NB = N_OUT // BR


def _tc_dense(self_x, sum_x, w1t, w2t, params):
    def body(self_ref, sum_ref, w1_ref, w2_ref, p_ref, out_ref, h_scr, acc_ref):
        ph = pl.program_id(0)
        blk = pl.program_id(1)

        @pl.when(ph == 0)
        def _phase0():
            h = (jnp.dot(self_ref[...], w1_ref[...],
                         preferred_element_type=jnp.float32)
                 + jnp.dot(sum_ref[...], w2_ref[...],
                           preferred_element_type=jnp.float32)
                 + p_ref[0:1, :])
            h = jnp.maximum(h, 0.0)
            h_scr[pl.ds(blk * BR, BR), :] = h

            @pl.when(blk == 0)
            def _init():
                acc_ref[...] = jnp.zeros_like(acc_ref)

            acc_ref[0:1, :] += jnp.sum(h, axis=0, keepdims=True)
            acc_ref[1:2, :] += jnp.sum(h * h, axis=0, keepdims=True)

        @pl.when(ph == 1)
        def _phase1():
            h = h_scr[pl.ds(blk * BR, BR), :]
            mean = acc_ref[0:1, :] * (1.0 / N_OUT)
            var = acc_ref[1:2, :] * (1.0 / N_OUT) - mean * mean
            y = (h - mean) * lax.rsqrt(var + 1e-5) * p_ref[1:2, :] + p_ref[2:3, :]
            nrm = jnp.sqrt(jnp.sum(y * y, axis=1, keepdims=True)) + 1e-6
            out_ref[...] = y / nrm

    return pl.pallas_call(
        body,
        grid=(2, NB),
        in_specs=[
            pl.BlockSpec((BR, D), lambda p, b_: (b_, 0)),
            pl.BlockSpec((BR, D), lambda p, b_: (b_, 0)),
            pl.BlockSpec((D, D), lambda p, b_: (0, 0)),
            pl.BlockSpec((D, D), lambda p, b_: (0, 0)),
            pl.BlockSpec((3, D), lambda p, b_: (0, 0)),
        ],
        out_specs=pl.BlockSpec((BR, D), lambda p, b_: (b_, 0)),
        out_shape=jax.ShapeDtypeStruct((N_OUT, D), jnp.float32),
        scratch_shapes=[
            pltpu.VMEM((N_OUT, D), jnp.float32),
            pltpu.VMEM((8, D), jnp.float32),
        ],
    )(self_x, sum_x, w1t, w2t, params)


def kernel(features, current, neigh_idx, W, b, gamma, beta):
    current = current.astype(jnp.int32)
    neigh = neigh_idx.astype(jnp.int32)
    idx26 = jnp.concatenate([current[:, None], neigh], axis=1)
    idx26 = jnp.pad(idx26, ((0, N_PAD - N_OUT), (0, 0)))
    idx_flat = idx26.reshape(-1)

    self_flat, sum_flat = _sc_gather(features, idx_flat)
    self_x = self_flat.reshape(N_PAD, D)[:N_OUT]
    sum_x = sum_flat.reshape(N_PAD, D)[:N_OUT]

    w1t = W[:, :D].T
    w2t = W[:, D:].T * (1.0 / S)
    params = jnp.stack([b, gamma, beta])
    return _tc_dense(self_x, sum_x, w1t, w2t, params)


# speedup vs baseline: 5.2277x; 1.5425x over previous
import functools

import numpy as np

import jax
import jax.numpy as jnp
from jax import lax
from jax.experimental import pallas as pl
from jax.experimental.pallas import tpu as pltpu
from jax.experimental.pallas import tpu_sc as plsc

N_TOTAL = 100000
N_OUT = 25000
D = 128
S = 25
R = S + 1
NC, NS = 2, 16
NW = NC * NS
BPW = 784
N_PAD = NW * BPW
C = 14
NCHUNK = BPW // C
CIDX = R * C
CSTR = 368


def _sc_gather(features, idx_packed):
    mesh = plsc.VectorSubcoreMesh(
        core_axis_name="c", subcore_axis_name="s",
        num_cores=NC, num_subcores=NS)

    @functools.partial(
        pl.kernel,
        out_type=(jax.ShapeDtypeStruct((N_PAD * D,), jnp.float32),
                  jax.ShapeDtypeStruct((N_PAD * D,), jnp.float32)),
        mesh=mesh,
        scratch_types=[
            pltpu.VMEM((NCHUNK * CSTR,), jnp.int32),
            pltpu.VMEM((CIDX, D), jnp.float32),
            pltpu.VMEM((CIDX, D), jnp.float32),
            pltpu.VMEM((C * D,), jnp.float32),
            pltpu.VMEM((C * D,), jnp.float32),
            pltpu.VMEM((C * D,), jnp.float32),
            pltpu.VMEM((C * D,), jnp.float32),
            pltpu.SemaphoreType.DMA,
            pltpu.SemaphoreType.DMA,
            pltpu.SemaphoreType.DMA,
            pltpu.SemaphoreType.DMA,
        ],
    )
    def k(feat_hbm, idx_hbm, self_hbm, sum_hbm,
          idx_all, gbuf0, gbuf1, sbuf0, sbuf1, mbuf0, mbuf1,
          gsem0, gsem1, osem0, osem1):
        wid = lax.axis_index("s") * NC + lax.axis_index("c")
        base = wid * BPW

        gbufs = (gbuf0, gbuf1)
        sbufs = (sbuf0, sbuf1)
        mbufs = (mbuf0, mbuf1)
        gsems = (gsem0, gsem1)
        osems = (osem0, osem1)

        def start_gather(kc, par):
            pltpu.async_copy(
                feat_hbm.at[idx_all.at[pl.ds(kc * CSTR, CIDX)]],
                gbufs[par], gsems[par])

        def consume(kc, par, first):
            gbuf, sbuf, mbuf = gbufs[par], sbufs[par], mbufs[par]
            pltpu.make_async_copy(
                feat_hbm.at[idx_all.at[pl.ds(0, CIDX)]], gbuf,
                gsems[par]).wait()
            if not first:
                pltpu.make_async_copy(
                    sbuf, self_hbm.at[pl.ds(0, C * D)], osems[par]).wait()
                pltpu.make_async_copy(
                    mbuf, sum_hbm.at[pl.ds(0, C * D)], osems[par]).wait()

            def node(c, carry2):
                r0 = c * R
                for u in range(8):
                    sbuf[pl.ds(c * D + 16 * u, 16)] = gbuf[r0, pl.ds(16 * u, 16)]
                accs = [gbuf[r0 + 1, pl.ds(16 * u, 16)] for u in range(8)]
                for j in range(2, R):
                    for u in range(8):
                        accs[u] = accs[u] + gbuf[r0 + j, pl.ds(16 * u, 16)]
                for u in range(8):
                    mbuf[pl.ds(c * D + 16 * u, 16)] = accs[u]
                return carry2

            lax.fori_loop(0, C, node, 0)
            c0 = base + kc * C
            pltpu.async_copy(sbuf, self_hbm.at[pl.ds(c0 * D, C * D)], osems[par])
            pltpu.async_copy(mbuf, sum_hbm.at[pl.ds(c0 * D, C * D)], osems[par])

        pltpu.sync_copy(idx_hbm.at[pl.ds(wid * NCHUNK * CSTR, NCHUNK * CSTR)],
                        idx_all)
        start_gather(0, 0)
        start_gather(1, 1)
        consume(0, 0, True)
        start_gather(2, 0)
        consume(1, 1, True)

        @pl.loop(1, NCHUNK // 2)
        def pair(p):
            k0 = 2 * p
            start_gather(k0 + 1, 1)
            consume(k0, 0, False)

            @pl.when(k0 + 2 < NCHUNK)
            def _pref():
                start_gather(k0 + 2, 0)

            consume(k0 + 1, 1, False)

        for par in range(2):
            pltpu.make_async_copy(
                sbufs[par], self_hbm.at[pl.ds(0, C * D)], osems[par]).wait()
            pltpu.make_async_copy(
                mbufs[par], sum_hbm.at[pl.ds(0, C * D)], osems[par]).wait()

    return k(features, idx_packed)


BR = 1000
NB = N_OUT // BR


def _tc_dense(self_x, sum_x, w1t, w2t, params):
    def body(self_ref, sum_ref, w1_ref, w2_ref, p_ref, out_ref, h_scr, acc_ref):
        ph = pl.program_id(0)
        blk = pl.program_id(1)

        @pl.when(ph == 0)
        def _phase0():
            h = (jnp.dot(self_ref[...], w1_ref[...],
                         preferred_element_type=jnp.float32)
                 + jnp.dot(sum_ref[...], w2_ref[...],
                           preferred_element_type=jnp.float32)
                 + p_ref[0:1, :])
            h = jnp.maximum(h, 0.0)
            h_scr[pl.ds(blk * BR, BR), :] = h

            @pl.when(blk == 0)
            def _init():
                acc_ref[...] = jnp.zeros_like(acc_ref)

            acc_ref[0:1, :] += jnp.sum(h, axis=0, keepdims=True)
            acc_ref[1:2, :] += jnp.sum(h * h, axis=0, keepdims=True)

        @pl.when(ph == 1)
        def _phase1():
            h = h_scr[pl.ds(blk * BR, BR), :]
            mean = acc_ref[0:1, :] * (1.0 / N_OUT)
            var = acc_ref[1:2, :] * (1.0 / N_OUT) - mean * mean
            y = (h - mean) * lax.rsqrt(var + 1e-5) * p_ref[1:2, :] + p_ref[2:3, :]
            nrm = jnp.sqrt(jnp.sum(y * y, axis=1, keepdims=True)) + 1e-6
            out_ref[...] = y / nrm

    return pl.pallas_call(
        body,
        grid=(2, NB),
        in_specs=[
            pl.BlockSpec((BR, D), lambda p, b_: (b_, 0)),
            pl.BlockSpec((BR, D), lambda p, b_: (b_, 0)),
            pl.BlockSpec((D, D), lambda p, b_: (0, 0)),
            pl.BlockSpec((D, D), lambda p, b_: (0, 0)),
            pl.BlockSpec((3, D), lambda p, b_: (0, 0)),
        ],
        out_specs=pl.BlockSpec((BR, D), lambda p, b_: (b_, 0)),
        out_shape=jax.ShapeDtypeStruct((N_OUT, D), jnp.float32),
        scratch_shapes=[
            pltpu.VMEM((N_OUT, D), jnp.float32),
            pltpu.VMEM((8, D), jnp.float32),
        ],
    )(self_x, sum_x, w1t, w2t, params)


def kernel(features, current, neigh_idx, W, b, gamma, beta):
    current = current.astype(jnp.int32)
    neigh = neigh_idx.astype(jnp.int32)
    idx26 = jnp.concatenate([current[:, None], neigh], axis=1)
    idx26 = jnp.pad(idx26, ((0, N_PAD - N_OUT), (0, 0)))
    idxp = idx26.reshape(NW, NCHUNK, C * R)
    idxp = jnp.pad(idxp, ((0, 0), (0, 0), (0, CSTR - CIDX)))
    idx_flat = idxp.reshape(-1)

    self_flat, sum_flat = _sc_gather(features, idx_flat)
    self_x = self_flat.reshape(N_PAD, D)[:N_OUT]
    sum_x = sum_flat.reshape(N_PAD, D)[:N_OUT]

    w1t = W[:, :D].T
    w2t = W[:, D:].T * (1.0 / S)
    params = jnp.stack([b, gamma, beta])
    return _tc_dense(self_x, sum_x, w1t, w2t, params)


# speedup vs baseline: 5.3599x; 1.0253x over previous
import functools

import numpy as np

import jax
import jax.numpy as jnp
from jax import lax
from jax.experimental import pallas as pl
from jax.experimental.pallas import tpu as pltpu
from jax.experimental.pallas import tpu_sc as plsc

N_TOTAL = 100000
N_OUT = 25000
D = 128
S = 25
R = S + 1
NC, NS = 2, 16
NW = NC * NS
BPW = 784
N_PAD = NW * BPW
C = 14
NCHUNK = BPW // C
CIDX = R * C
CSTR = 368


def _sc_gather(features, idx_packed):
    mesh = plsc.VectorSubcoreMesh(
        core_axis_name="c", subcore_axis_name="s",
        num_cores=NC, num_subcores=NS)

    @functools.partial(
        pl.kernel,
        out_type=(jax.ShapeDtypeStruct((N_PAD * D,), jnp.float32),
                  jax.ShapeDtypeStruct((N_PAD * D,), jnp.float32)),
        mesh=mesh,
        scratch_types=[
            pltpu.VMEM((NCHUNK * CSTR,), jnp.int32),
            pltpu.VMEM((CIDX, D), jnp.float32),
            pltpu.VMEM((CIDX, D), jnp.float32),
            pltpu.VMEM((C * D,), jnp.float32),
            pltpu.VMEM((C * D,), jnp.float32),
            pltpu.VMEM((C * D,), jnp.float32),
            pltpu.VMEM((C * D,), jnp.float32),
            pltpu.SemaphoreType.DMA,
            pltpu.SemaphoreType.DMA,
            pltpu.SemaphoreType.DMA,
            pltpu.SemaphoreType.DMA,
        ],
    )
    def k(feat_hbm, idx_hbm, self_hbm, sum_hbm,
          idx_all, gbuf0, gbuf1, sbuf0, sbuf1, mbuf0, mbuf1,
          gsem0, gsem1, osem0, osem1):
        wid = lax.axis_index("s") * NC + lax.axis_index("c")
        base = wid * BPW

        gbufs = (gbuf0, gbuf1)
        sbufs = (sbuf0, sbuf1)
        mbufs = (mbuf0, mbuf1)
        gsems = (gsem0, gsem1)
        osems = (osem0, osem1)

        def start_gather(kc, par):
            pltpu.async_copy(
                feat_hbm.at[idx_all.at[pl.ds(kc * CSTR, CIDX)]],
                gbufs[par], gsems[par])

        def consume(kc, par, first):
            gbuf, sbuf, mbuf = gbufs[par], sbufs[par], mbufs[par]
            pltpu.make_async_copy(
                feat_hbm.at[idx_all.at[pl.ds(0, CIDX)]], gbuf,
                gsems[par]).wait()
            if not first:
                pltpu.make_async_copy(
                    sbuf, self_hbm.at[pl.ds(0, C * D)], osems[par]).wait()
                pltpu.make_async_copy(
                    mbuf, sum_hbm.at[pl.ds(0, C * D)], osems[par]).wait()

            def node(c, carry2):
                r0 = c * R
                for u in range(8):
                    sbuf[pl.ds(c * D + 16 * u, 16)] = gbuf[r0, pl.ds(16 * u, 16)]
                accs = [gbuf[r0 + 1, pl.ds(16 * u, 16)] for u in range(8)]
                for j in range(2, R):
                    for u in range(8):
                        accs[u] = accs[u] + gbuf[r0 + j, pl.ds(16 * u, 16)]
                for u in range(8):
                    mbuf[pl.ds(c * D + 16 * u, 16)] = accs[u]
                return carry2

            lax.fori_loop(0, C, node, 0, unroll=2)
            c0 = base + kc * C
            pltpu.async_copy(sbuf, self_hbm.at[pl.ds(c0 * D, C * D)], osems[par])
            pltpu.async_copy(mbuf, sum_hbm.at[pl.ds(c0 * D, C * D)], osems[par])

        pltpu.sync_copy(idx_hbm.at[pl.ds(wid * NCHUNK * CSTR, NCHUNK * CSTR)],
                        idx_all)
        start_gather(0, 0)
        start_gather(1, 1)
        consume(0, 0, True)
        start_gather(2, 0)
        consume(1, 1, True)

        @pl.loop(1, NCHUNK // 2)
        def pair(p):
            k0 = 2 * p
            start_gather(k0 + 1, 1)
            consume(k0, 0, False)

            @pl.when(k0 + 2 < NCHUNK)
            def _pref():
                start_gather(k0 + 2, 0)

            consume(k0 + 1, 1, False)

        for par in range(2):
            pltpu.make_async_copy(
                sbufs[par], self_hbm.at[pl.ds(0, C * D)], osems[par]).wait()
            pltpu.make_async_copy(
                mbufs[par], sum_hbm.at[pl.ds(0, C * D)], osems[par]).wait()

    return k(features, idx_packed)


BR = 896
NB = N_PAD // BR


def _tc_dense(self_x, sum_x, w1t, w2t, params):
    def body(self_ref, sum_ref, w1_ref, w2_ref, p_ref, out_ref, h_scr, acc_ref):
        ph = pl.program_id(0)
        blk = pl.program_id(1)

        @pl.when(ph == 0)
        def _phase0():
            h = (jnp.dot(self_ref[...], w1_ref[...],
                         preferred_element_type=jnp.float32)
                 + jnp.dot(sum_ref[...], w2_ref[...],
                           preferred_element_type=jnp.float32)
                 + p_ref[0:1, :])
            h = jnp.maximum(h, 0.0)
            h_scr[pl.ds(blk * BR, BR), :] = h

            @pl.when(blk == 0)
            def _init():
                acc_ref[...] = jnp.zeros_like(acc_ref)

            rows = blk * BR + lax.broadcasted_iota(jnp.int32, (BR, 1), 0)
            hm = jnp.where(rows < N_OUT, h, 0.0)
            acc_ref[0:1, :] += jnp.sum(hm, axis=0, keepdims=True)
            acc_ref[1:2, :] += jnp.sum(hm * hm, axis=0, keepdims=True)

        @pl.when(ph == 1)
        def _phase1():
            h = h_scr[pl.ds(blk * BR, BR), :]
            mean = acc_ref[0:1, :] * (1.0 / N_OUT)
            var = acc_ref[1:2, :] * (1.0 / N_OUT) - mean * mean
            y = (h - mean) * lax.rsqrt(var + 1e-5) * p_ref[1:2, :] + p_ref[2:3, :]
            nrm = jnp.sqrt(jnp.sum(y * y, axis=1, keepdims=True)) + 1e-6
            out_ref[...] = y / nrm

    return pl.pallas_call(
        body,
        grid=(2, NB),
        in_specs=[
            pl.BlockSpec((BR, D), lambda p, b_: (b_, 0)),
            pl.BlockSpec((BR, D), lambda p, b_: (b_, 0)),
            pl.BlockSpec((D, D), lambda p, b_: (0, 0)),
            pl.BlockSpec((D, D), lambda p, b_: (0, 0)),
            pl.BlockSpec((3, D), lambda p, b_: (0, 0)),
        ],
        out_specs=pl.BlockSpec((BR, D), lambda p, b_: (b_, 0)),
        out_shape=jax.ShapeDtypeStruct((N_OUT, D), jnp.float32),
        scratch_shapes=[
            pltpu.VMEM((N_PAD, D), jnp.float32),
            pltpu.VMEM((8, D), jnp.float32),
        ],
    )(self_x, sum_x, w1t, w2t, params)


def kernel(features, current, neigh_idx, W, b, gamma, beta):
    current = current.astype(jnp.int32)
    neigh = neigh_idx.astype(jnp.int32)
    idx26 = jnp.concatenate([current[:, None], neigh], axis=1)
    idx26 = jnp.pad(idx26, ((0, N_PAD - N_OUT), (0, 0)))
    idxp = idx26.reshape(NW, NCHUNK, C * R)
    idxp = jnp.pad(idxp, ((0, 0), (0, 0), (0, CSTR - CIDX)))
    idx_flat = idxp.reshape(-1)

    self_flat, sum_flat = _sc_gather(features, idx_flat)
    self_x = self_flat.reshape(N_PAD, D)
    sum_x = sum_flat.reshape(N_PAD, D)

    w1t = W[:, :D].T
    w2t = W[:, D:].T * (1.0 / S)
    params = jnp.stack([b, gamma, beta])
    return _tc_dense(self_x, sum_x, w1t, w2t, params)


# speedup vs baseline: 6.0777x; 1.1339x over previous
import functools

import numpy as np

import jax
import jax.numpy as jnp
from jax import lax
from jax.experimental import pallas as pl
from jax.experimental.pallas import tpu as pltpu
from jax.experimental.pallas import tpu_sc as plsc

N_TOTAL = 100000
N_OUT = 25000
D = 128
S = 25
R = S + 1
NC, NS = 2, 16
NW = NC * NS
C = 14
CIDX = R * C
CSTR = 368
ACH = 72
BCH = 40
A_NODES = NS * ACH * C
B_NODES = NS * BCH * C
N_PAD = A_NODES + B_NODES


def _sc_gather(features, idx_packed):
    mesh = plsc.VectorSubcoreMesh(
        core_axis_name="c", subcore_axis_name="s",
        num_cores=NC, num_subcores=NS)

    @functools.partial(
        pl.kernel,
        out_type=(jax.ShapeDtypeStruct((N_PAD * D,), jnp.float32),
                  jax.ShapeDtypeStruct((N_PAD * D,), jnp.float32)),
        mesh=mesh,
        scratch_types=[
            pltpu.VMEM((ACH * CSTR,), jnp.int32),
            pltpu.VMEM((CIDX, D), jnp.float32),
            pltpu.VMEM((CIDX, D), jnp.float32),
            pltpu.VMEM((C * D,), jnp.float32),
            pltpu.VMEM((C * D,), jnp.float32),
            pltpu.VMEM((C * D,), jnp.float32),
            pltpu.VMEM((C * D,), jnp.float32),
            pltpu.SemaphoreType.DMA,
            pltpu.SemaphoreType.DMA,
            pltpu.SemaphoreType.DMA,
            pltpu.SemaphoreType.DMA,
        ],
    )
    def k(feat_hbm, idx_hbm, self_hbm, sum_hbm,
          idx_all, gbuf0, gbuf1, sbuf0, sbuf1, mbuf0, mbuf1,
          gsem0, gsem1, osem0, osem1):
        cid = lax.axis_index("c")
        sid = lax.axis_index("s")
        is0 = cid == 0
        nch = jnp.where(is0, ACH, BCH)
        base = jnp.where(is0, sid * (ACH * C), A_NODES + sid * (BCH * C))
        ioff = jnp.where(is0, sid * ACH, NS * ACH + sid * BCH) * CSTR

        gbufs = (gbuf0, gbuf1)
        sbufs = (sbuf0, sbuf1)
        mbufs = (mbuf0, mbuf1)
        gsems = (gsem0, gsem1)
        osems = (osem0, osem1)

        def start_gather(kc, par):
            pltpu.async_copy(
                feat_hbm.at[idx_all.at[pl.ds(kc * CSTR, CIDX)]],
                gbufs[par], gsems[par])

        def consume(kc, par, first):
            gbuf, sbuf, mbuf = gbufs[par], sbufs[par], mbufs[par]
            pltpu.make_async_copy(
                feat_hbm.at[idx_all.at[pl.ds(0, CIDX)]], gbuf,
                gsems[par]).wait()
            if not first:
                pltpu.make_async_copy(
                    sbuf, self_hbm.at[pl.ds(0, C * D)], osems[par]).wait()
                pltpu.make_async_copy(
                    mbuf, sum_hbm.at[pl.ds(0, C * D)], osems[par]).wait()

            def node(c, carry2):
                r0 = c * R
                for u in range(8):
                    sbuf[pl.ds(c * D + 16 * u, 16)] = gbuf[r0, pl.ds(16 * u, 16)]
                for ub in range(2):
                    us = [4 * ub + u for u in range(4)]
                    accs = [gbuf[r0 + 1, pl.ds(16 * u, 16)] for u in us]
                    for j in range(2, R):
                        for i, u in enumerate(us):
                            accs[i] = accs[i] + gbuf[r0 + j, pl.ds(16 * u, 16)]
                    for i, u in enumerate(us):
                        mbuf[pl.ds(c * D + 16 * u, 16)] = accs[i]
                return carry2

            lax.fori_loop(0, C, node, 0)
            c0 = base + kc * C
            pltpu.async_copy(sbuf, self_hbm.at[pl.ds(c0 * D, C * D)], osems[par])
            pltpu.async_copy(mbuf, sum_hbm.at[pl.ds(c0 * D, C * D)], osems[par])

        pltpu.sync_copy(idx_hbm.at[pl.ds(ioff, ACH * CSTR)], idx_all)
        start_gather(0, 0)
        start_gather(1, 1)
        consume(0, 0, True)
        start_gather(2, 0)
        consume(1, 1, True)

        @pl.loop(1, nch // 2)
        def pair(p):
            k0 = 2 * p
            start_gather(k0 + 1, 1)
            consume(k0, 0, False)

            @pl.when(k0 + 2 < nch)
            def _pref():
                start_gather(k0 + 2, 0)

            consume(k0 + 1, 1, False)

        for par in range(2):
            pltpu.make_async_copy(
                sbufs[par], self_hbm.at[pl.ds(0, C * D)], osems[par]).wait()
            pltpu.make_async_copy(
                mbufs[par], sum_hbm.at[pl.ds(0, C * D)], osems[par]).wait()

    return k(features, idx_packed)


BR = 896
NB = N_PAD // BR


def _tc_dense(self_x, sum_x, w1t, w2t, params):
    def body(self_ref, sum_ref, w1_ref, w2_ref, p_ref, out_ref, h_scr, acc_ref):
        ph = pl.program_id(0)
        blk = pl.program_id(1)

        @pl.when(ph == 0)
        def _phase0():
            h = (jnp.dot(self_ref[...], w1_ref[...],
                         preferred_element_type=jnp.float32)
                 + jnp.dot(sum_ref[...], w2_ref[...],
                           preferred_element_type=jnp.float32)
                 + p_ref[0:1, :])
            h = jnp.maximum(h, 0.0)
            h_scr[pl.ds(blk * BR, BR), :] = h

            @pl.when(blk == 0)
            def _init():
                acc_ref[...] = jnp.zeros_like(acc_ref)

            rows = blk * BR + lax.broadcasted_iota(jnp.int32, (BR, 1), 0)
            hm = jnp.where(rows < N_OUT, h, 0.0)
            acc_ref[0:1, :] += jnp.sum(hm, axis=0, keepdims=True)
            acc_ref[1:2, :] += jnp.sum(hm * hm, axis=0, keepdims=True)

        @pl.when(ph == 1)
        def _phase1():
            h = h_scr[pl.ds(blk * BR, BR), :]
            mean = acc_ref[0:1, :] * (1.0 / N_OUT)
            var = acc_ref[1:2, :] * (1.0 / N_OUT) - mean * mean
            y = (h - mean) * lax.rsqrt(var + 1e-5) * p_ref[1:2, :] + p_ref[2:3, :]
            nrm = jnp.sqrt(jnp.sum(y * y, axis=1, keepdims=True)) + 1e-6
            out_ref[...] = y / nrm

    return pl.pallas_call(
        body,
        grid=(2, NB),
        in_specs=[
            pl.BlockSpec((BR, D), lambda p, b_: (b_, 0)),
            pl.BlockSpec((BR, D), lambda p, b_: (b_, 0)),
            pl.BlockSpec((D, D), lambda p, b_: (0, 0)),
            pl.BlockSpec((D, D), lambda p, b_: (0, 0)),
            pl.BlockSpec((3, D), lambda p, b_: (0, 0)),
        ],
        out_specs=pl.BlockSpec((BR, D), lambda p, b_: (b_, 0)),
        out_shape=jax.ShapeDtypeStruct((N_OUT, D), jnp.float32),
        scratch_shapes=[
            pltpu.VMEM((N_PAD, D), jnp.float32),
            pltpu.VMEM((8, D), jnp.float32),
        ],
    )(self_x, sum_x, w1t, w2t, params)


def kernel(features, current, neigh_idx, W, b, gamma, beta):
    current = current.astype(jnp.int32)
    neigh = neigh_idx.astype(jnp.int32)
    idx26 = jnp.concatenate([current[:, None], neigh], axis=1)
    idx26 = jnp.pad(idx26, ((0, N_PAD - N_OUT), (0, 0)))
    pad_ch = ((0, 0), (0, 0), (0, CSTR - CIDX))
    p0 = jnp.pad(idx26[:A_NODES].reshape(NS, ACH, C * R), pad_ch)
    p1 = jnp.pad(idx26[A_NODES:].reshape(NS, BCH, C * R), pad_ch)
    idx_flat = jnp.concatenate(
        [p0.reshape(-1), p1.reshape(-1),
         jnp.zeros(((ACH - BCH) * CSTR,), jnp.int32)])

    self_flat, sum_flat = _sc_gather(features, idx_flat)
    self_x = self_flat.reshape(N_PAD, D)
    sum_x = sum_flat.reshape(N_PAD, D)

    w1t = W[:, :D].T
    w2t = W[:, D:].T * (1.0 / S)
    params = jnp.stack([b, gamma, beta])
    return _tc_dense(self_x, sum_x, w1t, w2t, params)


# speedup vs baseline: 6.2451x; 1.0275x over previous
import functools

import numpy as np

import jax
import jax.numpy as jnp
from jax import lax
from jax.experimental import pallas as pl
from jax.experimental.pallas import tpu as pltpu
from jax.experimental.pallas import tpu_sc as plsc

N_TOTAL = 100000
N_OUT = 25000
D = 128
S = 25
R = S + 1
NC, NS = 2, 16
NW = NC * NS
C = 14
CIDX = R * C
CSTR = 368
ACH = 80
BCH = 32
A_NODES = NS * ACH * C
B_NODES = NS * BCH * C
N_PAD = A_NODES + B_NODES


def _sc_gather(features, idx_packed):
    mesh = plsc.VectorSubcoreMesh(
        core_axis_name="c", subcore_axis_name="s",
        num_cores=NC, num_subcores=NS)

    @functools.partial(
        pl.kernel,
        out_type=(jax.ShapeDtypeStruct((N_PAD * D,), jnp.float32),
                  jax.ShapeDtypeStruct((N_PAD * D,), jnp.float32)),
        mesh=mesh,
        scratch_types=[
            pltpu.VMEM((ACH * CSTR,), jnp.int32),
            pltpu.VMEM((CIDX, D), jnp.float32),
            pltpu.VMEM((CIDX, D), jnp.float32),
            pltpu.VMEM((C * D,), jnp.float32),
            pltpu.VMEM((C * D,), jnp.float32),
            pltpu.VMEM((C * D,), jnp.float32),
            pltpu.VMEM((C * D,), jnp.float32),
            pltpu.SemaphoreType.DMA,
            pltpu.SemaphoreType.DMA,
            pltpu.SemaphoreType.DMA,
            pltpu.SemaphoreType.DMA,
        ],
    )
    def k(feat_hbm, idx_hbm, self_hbm, sum_hbm,
          idx_all, gbuf0, gbuf1, sbuf0, sbuf1, mbuf0, mbuf1,
          gsem0, gsem1, osem0, osem1):
        cid = lax.axis_index("c")
        sid = lax.axis_index("s")
        is0 = cid == 0
        nch = jnp.where(is0, ACH, BCH)
        base = jnp.where(is0, sid * (ACH * C), A_NODES + sid * (BCH * C))
        ioff = jnp.where(is0, sid * ACH, NS * ACH + sid * BCH) * CSTR

        gbufs = (gbuf0, gbuf1)
        sbufs = (sbuf0, sbuf1)
        mbufs = (mbuf0, mbuf1)
        gsems = (gsem0, gsem1)
        osems = (osem0, osem1)

        def start_gather(kc, par):
            pltpu.async_copy(
                feat_hbm.at[idx_all.at[pl.ds(kc * CSTR, CIDX)]],
                gbufs[par], gsems[par])

        def consume(kc, par, first):
            gbuf, sbuf, mbuf = gbufs[par], sbufs[par], mbufs[par]
            pltpu.make_async_copy(
                feat_hbm.at[idx_all.at[pl.ds(0, CIDX)]], gbuf,
                gsems[par]).wait()
            if not first:
                pltpu.make_async_copy(
                    sbuf, self_hbm.at[pl.ds(0, C * D)], osems[par]).wait()
                pltpu.make_async_copy(
                    mbuf, sum_hbm.at[pl.ds(0, C * D)], osems[par]).wait()

            def node(c, carry2):
                r0 = c * R
                for u in range(8):
                    sbuf[pl.ds(c * D + 16 * u, 16)] = gbuf[r0, pl.ds(16 * u, 16)]
                for ub in range(2):
                    us = [4 * ub + u for u in range(4)]
                    accs = [gbuf[r0 + 1, pl.ds(16 * u, 16)] for u in us]
                    for j in range(2, R):
                        for i, u in enumerate(us):
                            accs[i] = accs[i] + gbuf[r0 + j, pl.ds(16 * u, 16)]
                    for i, u in enumerate(us):
                        mbuf[pl.ds(c * D + 16 * u, 16)] = accs[i]
                return carry2

            lax.fori_loop(0, C, node, 0, unroll=2)
            c0 = base + kc * C
            pltpu.async_copy(sbuf, self_hbm.at[pl.ds(c0 * D, C * D)], osems[par])
            pltpu.async_copy(mbuf, sum_hbm.at[pl.ds(c0 * D, C * D)], osems[par])

        pltpu.sync_copy(idx_hbm.at[pl.ds(ioff, ACH * CSTR)], idx_all)
        start_gather(0, 0)
        start_gather(1, 1)
        consume(0, 0, True)
        start_gather(2, 0)
        consume(1, 1, True)

        @pl.loop(1, nch // 2)
        def pair(p):
            k0 = 2 * p
            start_gather(k0 + 1, 1)
            consume(k0, 0, False)

            @pl.when(k0 + 2 < nch)
            def _pref():
                start_gather(k0 + 2, 0)

            consume(k0 + 1, 1, False)

        for par in range(2):
            pltpu.make_async_copy(
                sbufs[par], self_hbm.at[pl.ds(0, C * D)], osems[par]).wait()
            pltpu.make_async_copy(
                mbufs[par], sum_hbm.at[pl.ds(0, C * D)], osems[par]).wait()

    return k(features, idx_packed)


BR = 896
NB = N_PAD // BR


def _tc_dense(self_x, sum_x, w1t, w2t, params):
    def body(self_ref, sum_ref, w1_ref, w2_ref, p_ref, out_ref, h_scr, acc_ref):
        ph = pl.program_id(0)
        blk = pl.program_id(1)

        @pl.when(ph == 0)
        def _phase0():
            h = (jnp.dot(self_ref[...].astype(jnp.bfloat16), w1_ref[...],
                         preferred_element_type=jnp.float32)
                 + jnp.dot(sum_ref[...].astype(jnp.bfloat16), w2_ref[...],
                           preferred_element_type=jnp.float32)
                 + p_ref[0:1, :])
            h = jnp.maximum(h, 0.0)
            h_scr[pl.ds(blk * BR, BR), :] = h

            @pl.when(blk == 0)
            def _init():
                acc_ref[...] = jnp.zeros_like(acc_ref)

            rows = blk * BR + lax.broadcasted_iota(jnp.int32, (BR, 1), 0)
            hm = jnp.where(rows < N_OUT, h, 0.0)
            acc_ref[0:1, :] += jnp.sum(hm, axis=0, keepdims=True)
            acc_ref[1:2, :] += jnp.sum(hm * hm, axis=0, keepdims=True)

        @pl.when(ph == 1)
        def _phase1():
            h = h_scr[pl.ds(blk * BR, BR), :]
            mean = acc_ref[0:1, :] * (1.0 / N_OUT)
            var = acc_ref[1:2, :] * (1.0 / N_OUT) - mean * mean
            y = (h - mean) * lax.rsqrt(var + 1e-5) * p_ref[1:2, :] + p_ref[2:3, :]
            nrm = jnp.sqrt(jnp.sum(y * y, axis=1, keepdims=True)) + 1e-6
            out_ref[...] = y / nrm

    return pl.pallas_call(
        body,
        grid=(2, NB),
        in_specs=[
            pl.BlockSpec((BR, D), lambda p, b_: (b_, 0)),
            pl.BlockSpec((BR, D), lambda p, b_: (b_, 0)),
            pl.BlockSpec((D, D), lambda p, b_: (0, 0)),
            pl.BlockSpec((D, D), lambda p, b_: (0, 0)),
            pl.BlockSpec((3, D), lambda p, b_: (0, 0)),
        ],
        out_specs=pl.BlockSpec((BR, D), lambda p, b_: (b_, 0)),
        out_shape=jax.ShapeDtypeStruct((N_OUT, D), jnp.float32),
        scratch_shapes=[
            pltpu.VMEM((N_PAD, D), jnp.float32),
            pltpu.VMEM((8, D), jnp.float32),
        ],
    )(self_x, sum_x, w1t, w2t, params)


def kernel(features, current, neigh_idx, W, b, gamma, beta):
    current = current.astype(jnp.int32)
    neigh = neigh_idx.astype(jnp.int32)
    idx26 = jnp.concatenate([current[:, None], neigh], axis=1)
    idx26 = jnp.pad(idx26, ((0, N_PAD - N_OUT), (0, 0)))
    pad_ch = ((0, 0), (0, 0), (0, CSTR - CIDX))
    p0 = jnp.pad(idx26[:A_NODES].reshape(NS, ACH, C * R), pad_ch)
    p1 = jnp.pad(idx26[A_NODES:].reshape(NS, BCH, C * R), pad_ch)
    idx_flat = jnp.concatenate(
        [p0.reshape(-1), p1.reshape(-1),
         jnp.zeros(((ACH - BCH) * CSTR,), jnp.int32)])

    self_flat, sum_flat = _sc_gather(features, idx_flat)
    self_x = self_flat.reshape(N_PAD, D)
    sum_x = sum_flat.reshape(N_PAD, D)

    w1t = W[:, :D].T.astype(jnp.bfloat16)
    w2t = (W[:, D:].T * (1.0 / S)).astype(jnp.bfloat16)
    params = jnp.stack([b, gamma, beta])
    return _tc_dense(self_x, sum_x, w1t, w2t, params)


# speedup vs baseline: 6.4118x; 1.0267x over previous
import functools

import numpy as np

import jax
import jax.numpy as jnp
from jax import lax
from jax.experimental import pallas as pl
from jax.experimental.pallas import tpu as pltpu
from jax.experimental.pallas import tpu_sc as plsc

N_TOTAL = 100000
N_OUT = 25000
D = 128
S = 25
R = S + 1
NC, NS = 2, 16
NW = NC * NS
C = 14
CIDX = R * C
CSTR = 368
ACH = 88
BCH = 24
A_NODES = NS * ACH * C
B_NODES = NS * BCH * C
N_PAD = A_NODES + B_NODES


def _sc_gather(features, idx_packed):
    mesh = plsc.VectorSubcoreMesh(
        core_axis_name="c", subcore_axis_name="s",
        num_cores=NC, num_subcores=NS)

    @functools.partial(
        pl.kernel,
        out_type=(jax.ShapeDtypeStruct((N_PAD * D,), jnp.float32),
                  jax.ShapeDtypeStruct((N_PAD * D,), jnp.float32)),
        mesh=mesh,
        scratch_types=[
            pltpu.VMEM((CSTR,), jnp.int32),
            pltpu.VMEM((CSTR,), jnp.int32),
            pltpu.VMEM((CIDX, D), jnp.float32),
            pltpu.VMEM((CIDX, D), jnp.float32),
            pltpu.VMEM((C * D,), jnp.float32),
            pltpu.VMEM((C * D,), jnp.float32),
            pltpu.VMEM((C * D,), jnp.float32),
            pltpu.VMEM((C * D,), jnp.float32),
            pltpu.SemaphoreType.DMA,
            pltpu.SemaphoreType.DMA,
            pltpu.SemaphoreType.DMA,
            pltpu.SemaphoreType.DMA,
            pltpu.SemaphoreType.DMA,
            pltpu.SemaphoreType.DMA,
        ],
    )
    def k(feat_hbm, idx_hbm, self_hbm, sum_hbm,
          ibuf0, ibuf1, gbuf0, gbuf1, sbuf0, sbuf1, mbuf0, mbuf1,
          isem0, isem1, gsem0, gsem1, osem0, osem1):
        cid = lax.axis_index("c")
        sid = lax.axis_index("s")
        is0 = cid == 0
        nch = jnp.where(is0, ACH, BCH)
        base = jnp.where(is0, sid * (ACH * C), A_NODES + sid * (BCH * C))
        ioff = jnp.where(is0, sid * ACH, NS * ACH + sid * BCH)

        ibufs = (ibuf0, ibuf1)
        gbufs = (gbuf0, gbuf1)
        sbufs = (sbuf0, sbuf1)
        mbufs = (mbuf0, mbuf1)
        isems = (isem0, isem1)
        gsems = (gsem0, gsem1)
        osems = (osem0, osem1)
        H0 = 184
        H1 = CIDX - H0

        def start_idx(kc, par):
            pltpu.async_copy(
                idx_hbm.at[pl.ds((ioff + kc) * CSTR, CSTR)],
                ibufs[par], isems[par])

        def start_gather(kc, par):
            ibuf = ibufs[par]
            pltpu.make_async_copy(
                idx_hbm.at[pl.ds(0, CSTR)], ibuf, isems[par]).wait()
            pltpu.async_copy(
                feat_hbm.at[ibuf.at[pl.ds(0, H0)]],
                gbufs[par].at[pl.ds(0, H0), :], gsems[par])
            pltpu.async_copy(
                feat_hbm.at[ibuf.at[pl.ds(H0, H1)]],
                gbufs[par].at[pl.ds(H0, H1), :], gsems[par])

        def consume(kc, par, first):
            gbuf, sbuf, mbuf = gbufs[par], sbufs[par], mbufs[par]
            pltpu.make_async_copy(
                feat_hbm.at[ibufs[par].at[pl.ds(0, H0)]],
                gbuf.at[pl.ds(0, H0), :], gsems[par]).wait()
            pltpu.make_async_copy(
                feat_hbm.at[ibufs[par].at[pl.ds(H0, H1)]],
                gbuf.at[pl.ds(H0, H1), :], gsems[par]).wait()

            @pl.when(kc + 2 < nch)
            def _pref_idx():
                start_idx(kc + 2, par)

            if not first:
                pltpu.make_async_copy(
                    sbuf, self_hbm.at[pl.ds(0, C * D)], osems[par]).wait()
                pltpu.make_async_copy(
                    mbuf, sum_hbm.at[pl.ds(0, C * D)], osems[par]).wait()

            def node(c, carry2):
                r0 = c * R
                for u in range(8):
                    sbuf[pl.ds(c * D + 16 * u, 16)] = gbuf[r0, pl.ds(16 * u, 16)]
                for ub in range(2):
                    us = [4 * ub + u for u in range(4)]
                    accs = [gbuf[r0 + 1, pl.ds(16 * u, 16)] for u in us]
                    for j in range(2, R):
                        for i, u in enumerate(us):
                            accs[i] = accs[i] + gbuf[r0 + j, pl.ds(16 * u, 16)]
                    for i, u in enumerate(us):
                        mbuf[pl.ds(c * D + 16 * u, 16)] = accs[i]
                return carry2

            lax.fori_loop(0, C, node, 0, unroll=2)
            c0 = base + kc * C
            pltpu.async_copy(sbuf, self_hbm.at[pl.ds(c0 * D, C * D)], osems[par])
            pltpu.async_copy(mbuf, sum_hbm.at[pl.ds(c0 * D, C * D)], osems[par])

        start_idx(0, 0)
        start_idx(1, 1)
        start_gather(0, 0)
        start_gather(1, 1)
        consume(0, 0, True)
        start_gather(2, 0)
        consume(1, 1, True)

        @pl.loop(1, nch // 2)
        def pair(p):
            k0 = 2 * p
            start_gather(k0 + 1, 1)
            consume(k0, 0, False)

            @pl.when(k0 + 2 < nch)
            def _pref():
                start_gather(k0 + 2, 0)

            consume(k0 + 1, 1, False)

        for par in range(2):
            pltpu.make_async_copy(
                sbufs[par], self_hbm.at[pl.ds(0, C * D)], osems[par]).wait()
            pltpu.make_async_copy(
                mbufs[par], sum_hbm.at[pl.ds(0, C * D)], osems[par]).wait()

    return k(features, idx_packed)


BR = 896
NB = N_PAD // BR


def _tc_dense(self_x, sum_x, w1t, w2t, params):
    def body(self_ref, sum_ref, w1_ref, w2_ref, p_ref, out_ref, h_scr, acc_ref):
        ph = pl.program_id(0)
        blk = pl.program_id(1)

        @pl.when(ph == 0)
        def _phase0():
            h = (jnp.dot(self_ref[...].astype(jnp.bfloat16), w1_ref[...],
                         preferred_element_type=jnp.float32)
                 + jnp.dot(sum_ref[...].astype(jnp.bfloat16), w2_ref[...],
                           preferred_element_type=jnp.float32)
                 + p_ref[0:1, :])
            h = jnp.maximum(h, 0.0)
            h_scr[pl.ds(blk * BR, BR), :] = h

            @pl.when(blk == 0)
            def _init():
                acc_ref[...] = jnp.zeros_like(acc_ref)

            rows = blk * BR + lax.broadcasted_iota(jnp.int32, (BR, 1), 0)
            hm = jnp.where(rows < N_OUT, h, 0.0)
            acc_ref[0:1, :] += jnp.sum(hm, axis=0, keepdims=True)
            acc_ref[1:2, :] += jnp.sum(hm * hm, axis=0, keepdims=True)

        @pl.when(ph == 1)
        def _phase1():
            h = h_scr[pl.ds(blk * BR, BR), :]
            mean = acc_ref[0:1, :] * (1.0 / N_OUT)
            var = acc_ref[1:2, :] * (1.0 / N_OUT) - mean * mean
            y = (h - mean) * lax.rsqrt(var + 1e-5) * p_ref[1:2, :] + p_ref[2:3, :]
            nrm = jnp.sqrt(jnp.sum(y * y, axis=1, keepdims=True)) + 1e-6
            out_ref[...] = y / nrm

    return pl.pallas_call(
        body,
        grid=(2, NB),
        in_specs=[
            pl.BlockSpec((BR, D), lambda p, b_: (b_, 0)),
            pl.BlockSpec((BR, D), lambda p, b_: (b_, 0)),
            pl.BlockSpec((D, D), lambda p, b_: (0, 0)),
            pl.BlockSpec((D, D), lambda p, b_: (0, 0)),
            pl.BlockSpec((3, D), lambda p, b_: (0, 0)),
        ],
        out_specs=pl.BlockSpec((BR, D), lambda p, b_: (b_, 0)),
        out_shape=jax.ShapeDtypeStruct((N_OUT, D), jnp.float32),
        scratch_shapes=[
            pltpu.VMEM((N_PAD, D), jnp.float32),
            pltpu.VMEM((8, D), jnp.float32),
        ],
    )(self_x, sum_x, w1t, w2t, params)


def kernel(features, current, neigh_idx, W, b, gamma, beta):
    current = current.astype(jnp.int32)
    neigh = neigh_idx.astype(jnp.int32)
    idx26 = jnp.concatenate([current[:, None], neigh], axis=1)
    idx26 = jnp.pad(idx26, ((0, N_PAD - N_OUT), (0, 0)))
    pad_ch = ((0, 0), (0, 0), (0, CSTR - CIDX))
    p0 = jnp.pad(idx26[:A_NODES].reshape(NS, ACH, C * R), pad_ch)
    p1 = jnp.pad(idx26[A_NODES:].reshape(NS, BCH, C * R), pad_ch)
    idx_flat = jnp.concatenate([p0.reshape(-1), p1.reshape(-1)])

    self_flat, sum_flat = _sc_gather(features, idx_flat)
    self_x = self_flat.reshape(N_PAD, D)
    sum_x = sum_flat.reshape(N_PAD, D)

    w1t = W[:, :D].T.astype(jnp.bfloat16)
    w2t = (W[:, D:].T * (1.0 / S)).astype(jnp.bfloat16)
    params = jnp.stack([b, gamma, beta])
    return _tc_dense(self_x, sum_x, w1t, w2t, params)


# speedup vs baseline: 6.4205x; 1.0014x over previous
import functools

import numpy as np

import jax
import jax.numpy as jnp
from jax import lax
from jax.experimental import pallas as pl
from jax.experimental.pallas import tpu as pltpu
from jax.experimental.pallas import tpu_sc as plsc

N_TOTAL = 100000
N_OUT = 25000
D = 128
S = 25
R = S + 1
NC, NS = 2, 16
NW = NC * NS
C = 14
CIDX = R * C
CSTR = 368
ACH = 88
BCH = 24
A_NODES = NS * ACH * C
B_NODES = NS * BCH * C
N_PAD = A_NODES + B_NODES


def _sc_gather(features, idx_packed):
    mesh = plsc.VectorSubcoreMesh(
        core_axis_name="c", subcore_axis_name="s",
        num_cores=NC, num_subcores=NS)

    @functools.partial(
        pl.kernel,
        out_type=(jax.ShapeDtypeStruct((N_PAD * D,), jnp.float32),
                  jax.ShapeDtypeStruct((N_PAD * D,), jnp.float32)),
        mesh=mesh,
        scratch_types=[
            pltpu.VMEM((CSTR,), jnp.int32),
            pltpu.VMEM((CSTR,), jnp.int32),
            pltpu.VMEM((CIDX, D), jnp.float32),
            pltpu.VMEM((CIDX, D), jnp.float32),
            pltpu.VMEM((C * D,), jnp.float32),
            pltpu.VMEM((C * D,), jnp.float32),
            pltpu.VMEM((C * D,), jnp.float32),
            pltpu.VMEM((C * D,), jnp.float32),
            pltpu.SemaphoreType.DMA,
            pltpu.SemaphoreType.DMA,
            pltpu.SemaphoreType.DMA,
            pltpu.SemaphoreType.DMA,
            pltpu.SemaphoreType.DMA,
            pltpu.SemaphoreType.DMA,
        ],
    )
    def k(feat_hbm, idx_hbm, self_hbm, sum_hbm,
          ibuf0, ibuf1, gbuf0, gbuf1, sbuf0, sbuf1, mbuf0, mbuf1,
          isem0, isem1, gsem0, gsem1, osem0, osem1):
        cid = lax.axis_index("c")
        sid = lax.axis_index("s")
        is0 = cid == 0
        nch = jnp.where(is0, ACH, BCH)
        base = jnp.where(is0, sid * (ACH * C), A_NODES + sid * (BCH * C))
        ioff = jnp.where(is0, sid * ACH, NS * ACH + sid * BCH)

        ibufs = (ibuf0, ibuf1)
        gbufs = (gbuf0, gbuf1)
        sbufs = (sbuf0, sbuf1)
        mbufs = (mbuf0, mbuf1)
        isems = (isem0, isem1)
        gsems = (gsem0, gsem1)
        osems = (osem0, osem1)
        HS = ((0, 88), (88, 96), (184, 96), (280, 84))

        def start_idx(kc, par):
            pltpu.async_copy(
                idx_hbm.at[pl.ds((ioff + kc) * CSTR, CSTR)],
                ibufs[par], isems[par])

        def start_gather(kc, par):
            ibuf = ibufs[par]
            pltpu.make_async_copy(
                idx_hbm.at[pl.ds(0, CSTR)], ibuf, isems[par]).wait()
            for off, ln in HS:
                pltpu.async_copy(
                    feat_hbm.at[ibuf.at[pl.ds(off, ln)]],
                    gbufs[par].at[pl.ds(off, ln), :], gsems[par])

        def consume(kc, par, first):
            gbuf, sbuf, mbuf = gbufs[par], sbufs[par], mbufs[par]
            for off, ln in HS:
                pltpu.make_async_copy(
                    feat_hbm.at[ibufs[par].at[pl.ds(off, ln)]],
                    gbuf.at[pl.ds(off, ln), :], gsems[par]).wait()

            @pl.when(kc + 2 < nch)
            def _pref_idx():
                start_idx(kc + 2, par)

            if not first:
                pltpu.make_async_copy(
                    sbuf, self_hbm.at[pl.ds(0, C * D)], osems[par]).wait()
                pltpu.make_async_copy(
                    mbuf, sum_hbm.at[pl.ds(0, C * D)], osems[par]).wait()

            def node(c, carry2):
                r0 = c * R
                for u in range(8):
                    sbuf[pl.ds(c * D + 16 * u, 16)] = gbuf[r0, pl.ds(16 * u, 16)]
                for ub in range(2):
                    us = [4 * ub + u for u in range(4)]
                    accs = [gbuf[r0 + 1, pl.ds(16 * u, 16)] for u in us]
                    for j in range(2, R):
                        for i, u in enumerate(us):
                            accs[i] = accs[i] + gbuf[r0 + j, pl.ds(16 * u, 16)]
                    for i, u in enumerate(us):
                        mbuf[pl.ds(c * D + 16 * u, 16)] = accs[i]
                return carry2

            lax.fori_loop(0, C, node, 0, unroll=2)
            c0 = base + kc * C
            pltpu.async_copy(sbuf, self_hbm.at[pl.ds(c0 * D, C * D)], osems[par])
            pltpu.async_copy(mbuf, sum_hbm.at[pl.ds(c0 * D, C * D)], osems[par])

        start_idx(0, 0)
        start_idx(1, 1)
        start_gather(0, 0)
        start_gather(1, 1)
        consume(0, 0, True)
        start_gather(2, 0)
        consume(1, 1, True)

        @pl.loop(1, nch // 2)
        def pair(p):
            k0 = 2 * p
            start_gather(k0 + 1, 1)
            consume(k0, 0, False)

            @pl.when(k0 + 2 < nch)
            def _pref():
                start_gather(k0 + 2, 0)

            consume(k0 + 1, 1, False)

        for par in range(2):
            pltpu.make_async_copy(
                sbufs[par], self_hbm.at[pl.ds(0, C * D)], osems[par]).wait()
            pltpu.make_async_copy(
                mbufs[par], sum_hbm.at[pl.ds(0, C * D)], osems[par]).wait()

    return k(features, idx_packed)


BR = 896
NB = N_PAD // BR


def _tc_dense(self_x, sum_x, w1t, w2t, params):
    def body(self_ref, sum_ref, w1_ref, w2_ref, p_ref, out_ref, h_scr, acc_ref):
        ph = pl.program_id(0)
        blk = pl.program_id(1)

        @pl.when(ph == 0)
        def _phase0():
            h = (jnp.dot(self_ref[...].astype(jnp.bfloat16), w1_ref[...],
                         preferred_element_type=jnp.float32)
                 + jnp.dot(sum_ref[...].astype(jnp.bfloat16), w2_ref[...],
                           preferred_element_type=jnp.float32)
                 + p_ref[0:1, :])
            h = jnp.maximum(h, 0.0)
            h_scr[pl.ds(blk * BR, BR), :] = h

            @pl.when(blk == 0)
            def _init():
                acc_ref[...] = jnp.zeros_like(acc_ref)

            rows = blk * BR + lax.broadcasted_iota(jnp.int32, (BR, 1), 0)
            hm = jnp.where(rows < N_OUT, h, 0.0)
            acc_ref[0:1, :] += jnp.sum(hm, axis=0, keepdims=True)
            acc_ref[1:2, :] += jnp.sum(hm * hm, axis=0, keepdims=True)

        @pl.when(ph == 1)
        def _phase1():
            h = h_scr[pl.ds(blk * BR, BR), :]
            mean = acc_ref[0:1, :] * (1.0 / N_OUT)
            var = acc_ref[1:2, :] * (1.0 / N_OUT) - mean * mean
            y = (h - mean) * lax.rsqrt(var + 1e-5) * p_ref[1:2, :] + p_ref[2:3, :]
            nrm = jnp.sqrt(jnp.sum(y * y, axis=1, keepdims=True)) + 1e-6
            out_ref[...] = y / nrm

    return pl.pallas_call(
        body,
        grid=(2, NB),
        in_specs=[
            pl.BlockSpec((BR, D), lambda p, b_: (b_, 0)),
            pl.BlockSpec((BR, D), lambda p, b_: (b_, 0)),
            pl.BlockSpec((D, D), lambda p, b_: (0, 0)),
            pl.BlockSpec((D, D), lambda p, b_: (0, 0)),
            pl.BlockSpec((3, D), lambda p, b_: (0, 0)),
        ],
        out_specs=pl.BlockSpec((BR, D), lambda p, b_: (b_, 0)),
        out_shape=jax.ShapeDtypeStruct((N_OUT, D), jnp.float32),
        scratch_shapes=[
            pltpu.VMEM((N_PAD, D), jnp.float32),
            pltpu.VMEM((8, D), jnp.float32),
        ],
    )(self_x, sum_x, w1t, w2t, params)


def kernel(features, current, neigh_idx, W, b, gamma, beta):
    current = current.astype(jnp.int32)
    neigh = neigh_idx.astype(jnp.int32)
    idx26 = jnp.concatenate([current[:, None], neigh], axis=1)
    idx26 = jnp.pad(idx26, ((0, N_PAD - N_OUT), (0, 0)))
    pad_ch = ((0, 0), (0, 0), (0, CSTR - CIDX))
    p0 = jnp.pad(idx26[:A_NODES].reshape(NS, ACH, C * R), pad_ch)
    p1 = jnp.pad(idx26[A_NODES:].reshape(NS, BCH, C * R), pad_ch)
    idx_flat = jnp.concatenate([p0.reshape(-1), p1.reshape(-1)])

    self_flat, sum_flat = _sc_gather(features, idx_flat)
    self_x = self_flat.reshape(N_PAD, D)
    sum_x = sum_flat.reshape(N_PAD, D)

    w1t = W[:, :D].T.astype(jnp.bfloat16)
    w2t = (W[:, D:].T * (1.0 / S)).astype(jnp.bfloat16)
    params = jnp.stack([b, gamma, beta])
    return _tc_dense(self_x, sum_x, w1t, w2t, params)


# speedup vs baseline: 6.4293x; 1.0014x over previous
import functools

import numpy as np

import jax
import jax.numpy as jnp
from jax import lax
from jax.experimental import pallas as pl
from jax.experimental.pallas import tpu as pltpu
from jax.experimental.pallas import tpu_sc as plsc

N_TOTAL = 100000
N_OUT = 25000
D = 128
S = 25
R = S + 1
NC, NS = 2, 16
NW = NC * NS
C = 14
CIDX = R * C
CSTR = 368
ACH = 88
BCH = 24
A_NODES = NS * ACH * C
B_NODES = NS * BCH * C
N_PAD = A_NODES + B_NODES


def _sc_gather(features, idx_packed):
    mesh = plsc.VectorSubcoreMesh(
        core_axis_name="c", subcore_axis_name="s",
        num_cores=NC, num_subcores=NS)

    @functools.partial(
        pl.kernel,
        out_type=(jax.ShapeDtypeStruct((N_PAD * D,), jnp.float32),
                  jax.ShapeDtypeStruct((N_PAD * D,), jnp.float32)),
        mesh=mesh,
        scratch_types=[
            pltpu.VMEM((CSTR,), jnp.int32),
            pltpu.VMEM((CSTR,), jnp.int32),
            pltpu.VMEM((CIDX, D), jnp.float32),
            pltpu.VMEM((CIDX, D), jnp.float32),
            pltpu.VMEM((C * D,), jnp.float32),
            pltpu.VMEM((C * D,), jnp.float32),
            pltpu.VMEM((C * D,), jnp.float32),
            pltpu.VMEM((C * D,), jnp.float32),
            pltpu.SemaphoreType.DMA,
            pltpu.SemaphoreType.DMA,
            pltpu.SemaphoreType.DMA,
            pltpu.SemaphoreType.DMA,
            pltpu.SemaphoreType.DMA,
            pltpu.SemaphoreType.DMA,
        ],
    )
    def k(feat_hbm, idx_hbm, self_hbm, sum_hbm,
          ibuf0, ibuf1, gbuf0, gbuf1, sbuf0, sbuf1, mbuf0, mbuf1,
          isem0, isem1, gsem0, gsem1, osem0, osem1):
        cid = lax.axis_index("c")
        sid = lax.axis_index("s")
        is0 = cid == 0
        nch = jnp.where(is0, ACH, BCH)
        base = jnp.where(is0, sid * (ACH * C), A_NODES + sid * (BCH * C))
        ioff = jnp.where(is0, sid * ACH, NS * ACH + sid * BCH)

        ibufs = (ibuf0, ibuf1)
        gbufs = (gbuf0, gbuf1)
        sbufs = (sbuf0, sbuf1)
        mbufs = (mbuf0, mbuf1)
        isems = (isem0, isem1)
        gsems = (gsem0, gsem1)
        osems = (osem0, osem1)
        H0 = 184
        H1 = CIDX - H0

        def start_idx(kc, par):
            pltpu.async_copy(
                idx_hbm.at[pl.ds((ioff + kc) * CSTR, CSTR)],
                ibufs[par], isems[par])

        def start_gather(kc, par):
            ibuf = ibufs[par]
            pltpu.make_async_copy(
                idx_hbm.at[pl.ds(0, CSTR)], ibuf, isems[par]).wait()
            pltpu.async_copy(
                feat_hbm.at[ibuf.at[pl.ds(0, H0)]],
                gbufs[par].at[pl.ds(0, H0), :], gsems[par])
            pltpu.async_copy(
                feat_hbm.at[ibuf.at[pl.ds(H0, H1)]],
                gbufs[par].at[pl.ds(H0, H1), :], gsems[par])

        def consume(kc, par, first):
            gbuf, sbuf, mbuf = gbufs[par], sbufs[par], mbufs[par]
            pltpu.make_async_copy(
                feat_hbm.at[ibufs[par].at[pl.ds(0, H0)]],
                gbuf.at[pl.ds(0, H0), :], gsems[par]).wait()
            pltpu.make_async_copy(
                feat_hbm.at[ibufs[par].at[pl.ds(H0, H1)]],
                gbuf.at[pl.ds(H0, H1), :], gsems[par]).wait()

            @pl.when(kc + 2 < nch)
            def _pref_idx():
                start_idx(kc + 2, par)

            if not first:
                pltpu.make_async_copy(
                    sbuf, self_hbm.at[pl.ds(0, C * D)], osems[par]).wait()
                pltpu.make_async_copy(
                    mbuf, sum_hbm.at[pl.ds(0, C * D)], osems[par]).wait()

            def node(c, carry2):
                r0 = c * R
                for u in range(8):
                    sbuf[pl.ds(c * D + 16 * u, 16)] = gbuf[r0, pl.ds(16 * u, 16)]
                for ub in range(2):
                    us = [4 * ub + u for u in range(4)]
                    accs = [gbuf[r0 + 1, pl.ds(16 * u, 16)] for u in us]
                    for j in range(2, R):
                        for i, u in enumerate(us):
                            accs[i] = accs[i] + gbuf[r0 + j, pl.ds(16 * u, 16)]
                    for i, u in enumerate(us):
                        mbuf[pl.ds(c * D + 16 * u, 16)] = accs[i]
                return carry2

            lax.fori_loop(0, C, node, 0, unroll=2)
            c0 = base + kc * C
            pltpu.async_copy(sbuf, self_hbm.at[pl.ds(c0 * D, C * D)], osems[par])
            pltpu.async_copy(mbuf, sum_hbm.at[pl.ds(c0 * D, C * D)], osems[par])

        start_idx(0, 0)
        start_idx(1, 1)
        start_gather(0, 0)
        start_gather(1, 1)
        consume(0, 0, True)
        start_gather(2, 0)
        consume(1, 1, True)

        @pl.loop(1, nch // 2)
        def pair(p):
            k0 = 2 * p
            start_gather(k0 + 1, 1)
            consume(k0, 0, False)

            @pl.when(k0 + 2 < nch)
            def _pref():
                start_gather(k0 + 2, 0)

            consume(k0 + 1, 1, False)

        for par in range(2):
            pltpu.make_async_copy(
                sbufs[par], self_hbm.at[pl.ds(0, C * D)], osems[par]).wait()
            pltpu.make_async_copy(
                mbufs[par], sum_hbm.at[pl.ds(0, C * D)], osems[par]).wait()

    return k(features, idx_packed)


BR = 896
NB = N_PAD // BR


def _tc_dense(self_x, sum_x, w1t, w2t, params):
    def body(self_ref, sum_ref, w1_ref, w2_ref, p_ref, out_ref, h_scr, acc_ref):
        ph = pl.program_id(0)
        blk = pl.program_id(1)

        @pl.when(ph == 0)
        def _phase0():
            h = (jnp.dot(self_ref[...].astype(jnp.bfloat16), w1_ref[...],
                         preferred_element_type=jnp.float32)
                 + jnp.dot(sum_ref[...].astype(jnp.bfloat16), w2_ref[...],
                           preferred_element_type=jnp.float32)
                 + p_ref[0:1, :])
            h = jnp.maximum(h, 0.0)
            h_scr[pl.ds(blk * BR, BR), :] = h

            @pl.when(blk == 0)
            def _init():
                acc_ref[...] = jnp.zeros_like(acc_ref)

            rows = blk * BR + lax.broadcasted_iota(jnp.int32, (BR, 1), 0)
            hm = jnp.where(rows < N_OUT, h, 0.0)
            acc_ref[0:1, :] += jnp.sum(hm, axis=0, keepdims=True)
            acc_ref[1:2, :] += jnp.sum(hm * hm, axis=0, keepdims=True)

        @pl.when(ph == 1)
        def _phase1():
            h = h_scr[pl.ds(blk * BR, BR), :]
            mean = acc_ref[0:1, :] * (1.0 / N_OUT)
            var = acc_ref[1:2, :] * (1.0 / N_OUT) - mean * mean
            y = (h - mean) * lax.rsqrt(var + 1e-5) * p_ref[1:2, :] + p_ref[2:3, :]
            nrm = jnp.sqrt(jnp.sum(y * y, axis=1, keepdims=True)) + 1e-6
            out_ref[...] = y / nrm

    return pl.pallas_call(
        body,
        grid=(2, NB),
        in_specs=[
            pl.BlockSpec((BR, D), lambda p, b_: (b_, 0)),
            pl.BlockSpec((BR, D), lambda p, b_: (b_, 0)),
            pl.BlockSpec((D, D), lambda p, b_: (0, 0)),
            pl.BlockSpec((D, D), lambda p, b_: (0, 0)),
            pl.BlockSpec((3, D), lambda p, b_: (0, 0)),
        ],
        out_specs=pl.BlockSpec((BR, D), lambda p, b_: (b_, 0)),
        out_shape=jax.ShapeDtypeStruct((N_OUT, D), jnp.float32),
        scratch_shapes=[
            pltpu.VMEM((N_PAD, D), jnp.float32),
            pltpu.VMEM((8, D), jnp.float32),
        ],
    )(self_x, sum_x, w1t, w2t, params)


def kernel(features, current, neigh_idx, W, b, gamma, beta):
    current = current.astype(jnp.int32)
    neigh = neigh_idx.astype(jnp.int32)
    idx26 = jnp.concatenate([current[:, None], neigh], axis=1)
    idx26 = jnp.pad(idx26, ((0, N_PAD - N_OUT), (0, 0)))
    pad_ch = ((0, 0), (0, 0), (0, CSTR - CIDX))
    p0 = jnp.pad(idx26[:A_NODES].reshape(NS, ACH, C * R), pad_ch)
    p1 = jnp.pad(idx26[A_NODES:].reshape(NS, BCH, C * R), pad_ch)
    idx_flat = jnp.concatenate([p0.reshape(-1), p1.reshape(-1)])

    self_flat, sum_flat = _sc_gather(features, idx_flat)
    self_x = self_flat.reshape(N_PAD, D)
    sum_x = sum_flat.reshape(N_PAD, D)

    w1t = W[:, :D].T.astype(jnp.bfloat16)
    w2t = (W[:, D:].T * (1.0 / S)).astype(jnp.bfloat16)
    params = jnp.stack([b, gamma, beta])
    return _tc_dense(self_x, sum_x, w1t, w2t, params)
